# Initial kernel scaffold; baseline (speedup 1.0000x reference)
import jax
import jax.numpy as jnp
from jax.experimental import pallas as pl


def kernel(tokens, dispatch_weights, combine_weights, gate_W, value_W, out_W, out_scale):
    raise NotImplementedError("write your pallas kernel here")


# speedup vs baseline: 2.9108x; 2.9108x over previous
import functools

import jax
import jax.numpy as jnp
from jax.experimental import pallas as pl


def _gelu_exact(x):
    return 0.5 * x * (1.0 + jax.lax.erf(x * 0.7071067811865476))


def _moe_body(x_ref, fd_ref, fc_ref, gw_ref, vw_ref, ow_ref, out_ref, *, bt, d, h, hc):
    e = pl.program_id(1)

    @pl.when(e == 0)
    def _init():
        out_ref[...] = jnp.zeros_like(out_ref)

    x = x_ref[...]
    acc = jnp.zeros((bt, d), jnp.float32)
    for hi in range(h // hc):
        gw = gw_ref[0, hi * hc:(hi + 1) * hc, :]
        vw = vw_ref[0, hi * hc:(hi + 1) * hc, :]
        ow = ow_ref[0, :, hi * hc:(hi + 1) * hc]
        g = jax.lax.dot_general(x, gw, (((1,), (1,)), ((), ())),
                                preferred_element_type=jnp.float32)
        v = jax.lax.dot_general(x, vw, (((1,), (1,)), ((), ())),
                                preferred_element_type=jnp.float32)
        gv = _gelu_exact(g) * v
        acc = acc + jax.lax.dot_general(gv, ow, (((1,), (1,)), ((), ())),
                                        preferred_element_type=jnp.float32)
    fd = fd_ref[0, 0, :]
    fc = fc_ref[0, 0, :]
    w = jnp.where(fd > 0, fc, 0.0).reshape(bt, 1)
    out_ref[...] += acc * w


@jax.jit
def kernel(tokens, dispatch_weights, combine_weights, gate_W, value_W, out_W, out_scale):
    B, N, D = tokens.shape
    E = dispatch_weights.shape[-1]
    H = gate_W.shape[1]
    T = B * N
    BT = 1024
    HC = 512

    flat = tokens.reshape(T, D)
    fdT = dispatch_weights.reshape(T, E).T.reshape(E, 1, T)
    fcT = combine_weights.reshape(T, E).T.reshape(E, 1, T)
    ow_scaled = out_W * out_scale[:, None, None]

    nt = T // BT
    body = functools.partial(_moe_body, bt=BT, d=D, h=H, hc=HC)
    out = pl.pallas_call(
        body,
        grid=(nt, E),
        in_specs=[
            pl.BlockSpec((BT, D), lambda t, e: (t, 0)),
            pl.BlockSpec((1, 1, BT), lambda t, e: (e, 0, t)),
            pl.BlockSpec((1, 1, BT), lambda t, e: (e, 0, t)),
            pl.BlockSpec((1, H, D), lambda t, e: (e, 0, 0)),
            pl.BlockSpec((1, H, D), lambda t, e: (e, 0, 0)),
            pl.BlockSpec((1, D, H), lambda t, e: (e, 0, 0)),
        ],
        out_specs=pl.BlockSpec((BT, D), lambda t, e: (t, 0)),
        out_shape=jax.ShapeDtypeStruct((T, D), jnp.float32),
    )(flat, fdT, fcT, gate_W, value_W, ow_scaled)
    return out.reshape(B, N, D)
